# baseline (speedup 1.0000x reference)
import jax, jax.numpy as jnp

FM = 128
TEST_ROUNDS = 2
N_VARS = 10000
N_LITS = 2 * N_VARS
N_CLAUSES = 40000
N_GRAPHS = 64
NORM_EPS = 1e-6
EPS = 1e-8


def _relu6(x):
    return jnp.minimum(jnp.maximum(x, 0.0), 6.0)


def _mlp(p, x, do_layer_norm=True):
    n = len(p['W'])
    for i in range(n):
        x = x @ p['W'][i] + p['b'][i]
        if i < n - 1:
            x = _relu6(x)
            if do_layer_norm:
                m = jnp.mean(x, axis=-1, keepdims=True)
                v = jnp.var(x, axis=-1, keepdims=True)
                x = (x - m) * jax.lax.rsqrt(v + 1e-6)
    return x


def _norm0(x):
    m = jnp.mean(x, axis=0, keepdims=True)
    v = jnp.var(x, axis=0, keepdims=True)
    return (x - m) * jax.lax.rsqrt(v + NORM_EPS)


def _cl(dense_lit, lit_idx, clause_idx):
    return jax.ops.segment_sum(jnp.take(dense_lit, lit_idx, axis=0), clause_idx, num_segments=N_CLAUSES)


def _adj(dense_cl, lit_idx, clause_idx):
    return jax.ops.segment_sum(jnp.take(dense_cl, clause_idx, axis=0), lit_idx, num_segments=N_LITS)


def _sploss(q, lit_idx, clause_idx):
    lits = jnp.concatenate([q, -q], axis=0)
    lits = jax.nn.softplus(lits)
    cv = _cl(lits, lit_idx, clause_idx)
    return jnp.exp(-cv)


def kernel(lit_idx, clause_idx, clause_graph_ids, var_graph_ids, params):
    p = params
    L = jnp.ones((N_VARS, FM), jnp.float32) * p['L_init_scale']
    C = jnp.ones((N_CLAUSES, FM), jnp.float32) * p['C_init_scale']
    loss = jnp.array(0.0, jnp.float32)
    logits = jnp.zeros((N_VARS, 1), jnp.float32)
    for step in range(TEST_ROUNDS):
        lit1, lit2 = jnp.split(L, 2, axis=1)
        literals = jnp.concatenate([lit1, lit2], axis=0)
        LC_msgs = _cl(literals, lit_idx, clause_idx) * p['LC_scale']
        query = _mlp(p['Vq'], L, do_layer_norm=False)
        clauses_loss = _sploss(query, lit_idx, clause_idx)
        g = _adj(clauses_loss, lit_idx, clause_idx)
        g1, g2 = jnp.split(g, 2, axis=0)
        variables_grad = (-g1 * jax.nn.sigmoid(query) + g2 * jax.nn.sigmoid(-query)) * p['G_scale']
        C = _mlp(p['Cu'], jnp.concatenate([C, clauses_loss, LC_msgs], axis=-1))
        C = _norm0(C)
        CL_msgs = _adj(C, lit_idx, clause_idx) * p['CL_scale']
        CL1, CL2 = jnp.split(CL_msgs, 2, axis=0)
        L = _mlp(p['Lu'], jnp.concatenate([L, CL1, CL2, variables_grad], axis=-1))
        L = _norm0(L)
        logits = _mlp(p['Vs'], L)
        cv = _sploss(logits, lit_idx, clause_idx)
        per_clause_loss = cv * (-jnp.log(1.0 - cv + EPS))
        per_graph_loss = jax.ops.segment_sum(per_clause_loss, clause_graph_ids, num_segments=N_GRAPHS)
        loss = loss + jnp.sum(jnp.sqrt(per_graph_loss + 1e-6))
    return logits, loss / float(TEST_ROUNDS), jnp.array(TEST_ROUNDS - 1, jnp.int32)


# speedup vs baseline: 1.8901x; 1.8901x over previous
import functools

import jax
import jax.numpy as jnp
from jax import lax
from jax.experimental import pallas as pl
from jax.experimental.pallas import tpu as pltpu
from jax.experimental.pallas import tpu_sc as plsc

FM = 128
TEST_ROUNDS = 2
N_VARS = 10000
N_LITS = 2 * N_VARS
N_CLAUSES = 40000
N_EDGES = 120000
N_GRAPHS = 64
NORM_EPS = 1e-6
EPS = 1e-8

NC, NS = 2, 16
NW = NC * NS
EB = 128


def _sc_segsum(src, gidx, sidx, n_dst, name):
    G, n_src, CW = src.shape
    K = gidx.shape[1]
    n_acc = ((n_dst + 8) + NS * 8 - 1) // (NS * 8) * (NS * 8)
    zrows = n_acc // NS
    frows = (n_dst // (NS * 8)) * 8
    frem = n_dst - frows * NS
    gpc = (G + NC - 1) // NC
    zeros = jnp.zeros((zrows, CW), jnp.float32)
    mesh = plsc.VectorSubcoreMesh(core_axis_name="c", subcore_axis_name="s",
                                  num_cores=NC, num_subcores=NS)

    def body(src_h, gidx_h, sidx_h, zero_h, out_h,
             gidx_v, sidx_v, rows_v, zeros_v, acc_sh, sem):
        c = lax.axis_index("c")
        s = lax.axis_index("s")
        pltpu.sync_copy(gidx_h.at[s], gidx_v)
        pltpu.sync_copy(sidx_h.at[s], sidx_v)
        pltpu.sync_copy(zero_h, zeros_v)
        for gi in range(gpc):
            g = gi * NC + c

            @pl.when(g < G)
            def _():
                pltpu.sync_copy(zeros_v, acc_sh.at[pl.ds(s * zrows, zrows)])
                plsc.subcore_barrier()

                def chunk(j, carry):
                    pltpu.async_copy(src_h.at[g].at[gidx_v.at[j]], rows_v,
                                     sem).wait()
                    pltpu.sync_copy(rows_v, acc_sh.at[sidx_v.at[j]], add=True)
                    return carry

                lax.fori_loop(0, K, chunk, 0)
                plsc.subcore_barrier()
                if frows > 0:
                    pltpu.sync_copy(acc_sh.at[pl.ds(s * frows, frows)],
                                    out_h.at[g].at[pl.ds(s * frows, frows)])
                if frem > 0:
                    @pl.when(s == 0)
                    def _():
                        pltpu.sync_copy(
                            acc_sh.at[pl.ds(NS * frows, frem)],
                            out_h.at[g].at[pl.ds(NS * frows, frem)])
                plsc.subcore_barrier()

    return pl.kernel(
        body,
        out_type=jax.ShapeDtypeStruct((G, n_dst, CW), jnp.float32),
        mesh=mesh,
        scratch_types=[
            pltpu.VMEM((K, EB), jnp.int32),
            pltpu.VMEM((K, EB), jnp.int32),
            pltpu.VMEM((EB, CW), jnp.float32),
            pltpu.VMEM((zrows, CW), jnp.float32),
            pltpu.VMEM_SHARED((n_acc, CW), jnp.float32),
            pltpu.SemaphoreType.DMA,
        ],
        name=name,
        compiler_params=pltpu.CompilerParams(use_tc_tiling_on_sc=False),
    )(src, gidx, sidx, zeros)


def _relu6(x):
    return jnp.minimum(jnp.maximum(x, 0.0), 6.0)


def _rowln(x):
    m = jnp.mean(x, axis=-1, keepdims=True)
    xm = x - m
    v = jnp.mean(xm * xm, axis=-1, keepdims=True)
    return xm * lax.rsqrt(v + 1e-6)


def _softplus(x):
    return jnp.maximum(x, 0.0) + jnp.log1p(jnp.exp(-jnp.abs(x)))


def _dot(a, b):
    return jnp.dot(a, b, preferred_element_type=jnp.float32)


def _full(shape):
    return pl.BlockSpec(shape, lambda b: tuple(0 for _ in shape))


def _vq(L, W1, b1, W2, b2, W3, b3):
    RB = 2000
    nb = N_VARS // RB

    def body(x_ref, w1, bb1, w2, bb2, w3, bb3, q_ref, spp_ref, spn_ref):
        h = _relu6(_dot(x_ref[...], w1[...]) + bb1[...])
        h = _relu6(_dot(h, w2[...]) + bb2[...])
        q = _dot(h, w3[...]) + bb3[...]
        q_ref[...] = q
        spp = _softplus(q)
        spn = _softplus(-q)
        for g in range(8):
            spp_ref[g] = spp[:, g * 16:(g + 1) * 16]
            spn_ref[g] = spn[:, g * 16:(g + 1) * 16]

    return pl.pallas_call(
        body,
        grid=(nb,),
        in_specs=[pl.BlockSpec((RB, FM), lambda b: (b, 0)),
                  _full((FM, FM)), _full((1, FM)),
                  _full((FM, FM)), _full((1, FM)),
                  _full((FM, FM)), _full((1, FM))],
        out_specs=[pl.BlockSpec((RB, FM), lambda b: (b, 0)),
                   pl.BlockSpec((8, RB, 16), lambda b: (0, b, 0)),
                   pl.BlockSpec((8, RB, 16), lambda b: (0, b, 0))],
        out_shape=[jax.ShapeDtypeStruct((N_VARS, FM), jnp.float32),
                   jax.ShapeDtypeStruct((8, N_VARS, 16), jnp.float32),
                   jax.ShapeDtypeStruct((8, N_VARS, 16), jnp.float32)],
    )(L, W1, b1, W2, b2, W3, b3)


def _closs(cv8):
    RB = 2000
    nb = N_CLAUSES // RB

    def body(cv_ref, full_ref, sc_ref):
        y = jnp.exp(-jnp.concatenate([cv_ref[g] for g in range(8)], axis=1))
        full_ref[...] = y
        for k in range(4):
            sc_ref[k] = y[:, k * 32:(k + 1) * 32]

    return pl.pallas_call(
        body,
        grid=(nb,),
        in_specs=[pl.BlockSpec((8, RB, 16), lambda b: (0, b, 0))],
        out_specs=[pl.BlockSpec((RB, FM), lambda b: (b, 0)),
                   pl.BlockSpec((4, RB, 32), lambda b: (0, b, 0))],
        out_shape=[jax.ShapeDtypeStruct((N_CLAUSES, FM), jnp.float32),
                   jax.ShapeDtypeStruct((4, N_CLAUSES, 32), jnp.float32)],
    )(cv8)


def _cu(C, closs, lc, lc_scale, W1, b1, W2, b2, W3, b3):
    RB = 2000
    nb = N_CLAUSES // RB

    def body(c_ref, cl_ref, lc_ref, sc, w1, bb1, w2, bb2,
             w3, bb3, y_ref, st_ref, acc):
        i = pl.program_id(0)
        xin = jnp.concatenate(
            [c_ref[...], cl_ref[...], lc_ref[...] * sc[0, 0]], axis=1)
        h = _dot(xin, w1[...]) + bb1[...]
        h = _rowln(_relu6(h))
        h = _rowln(_relu6(_dot(h, w2[...]) + bb2[...]))
        y = _dot(h, w3[...]) + bb3[...]
        y_ref[...] = y
        ps = jnp.sum(y, axis=0, keepdims=True)
        pq = jnp.sum(y * y, axis=0, keepdims=True)

        @pl.when(i == 0)
        def _():
            acc[0:1] = ps
            acc[1:2] = pq

        @pl.when(i > 0)
        def _():
            acc[0:1] += ps
            acc[1:2] += pq

        st_ref[...] = acc[...]

    return pl.pallas_call(
        body,
        grid=(nb,),
        in_specs=[pl.BlockSpec((RB, FM), lambda b: (b, 0)),
                  pl.BlockSpec((RB, FM), lambda b: (b, 0)),
                  pl.BlockSpec((RB, 64), lambda b: (b, 0)),
                  _full((1, 1)),
                  _full((320, 256)), _full((1, 256)),
                  _full((256, 256)), _full((1, 256)),
                  _full((256, FM)), _full((1, FM))],
        out_specs=[pl.BlockSpec((RB, FM), lambda b: (b, 0)),
                   _full((2, FM))],
        out_shape=[jax.ShapeDtypeStruct((N_CLAUSES, FM), jnp.float32),
                   jax.ShapeDtypeStruct((2, FM), jnp.float32)],
        scratch_shapes=[pltpu.VMEM((2, FM), jnp.float32)],
    )(C, closs, lc, lc_scale, W1, b1, W2, b2, W3, b3)


def _apply_norm(x, stats, emit_sc, name):
    n_rows = x.shape[0]
    RB = 2000
    nb = n_rows // RB
    fn = float(n_rows)

    def body(x_ref, st, y_ref, *maybe_sc):
        m = st[0:1] / fn
        ex2 = st[1:2] / fn
        v = ex2 - m * m
        y = (x_ref[...] - m) * lax.rsqrt(v + NORM_EPS)
        y_ref[...] = y
        if emit_sc:
            for k in range(4):
                maybe_sc[0][k] = y[:, k * 32:(k + 1) * 32]

    out_specs = [pl.BlockSpec((RB, FM), lambda b: (b, 0))]
    out_shape = [jax.ShapeDtypeStruct((n_rows, FM), jnp.float32)]
    if emit_sc:
        out_specs.append(pl.BlockSpec((4, RB, 32), lambda b: (0, b, 0)))
        out_shape.append(jax.ShapeDtypeStruct((4, n_rows, 32), jnp.float32))
    return pl.pallas_call(
        body,
        grid=(nb,),
        in_specs=[pl.BlockSpec((RB, FM), lambda b: (b, 0)), _full((2, FM))],
        out_specs=out_specs,
        out_shape=out_shape,
        name=name,
    )(x, stats)


def _lu(L, c1, c2, g1, g2, q, cl_scale, g_scale,
        W1, b1, W2, b2, W3, b3):
    RB = 2000
    nb = N_VARS // RB

    def body(l_ref, c1_ref, c2_ref, g1_ref, g2_ref, q_ref, cs, gs,
             w1, bb1, w2, bb2, w3, bb3,
             y_ref, st_ref, acc):
        i = pl.program_id(0)
        qv = q_ref[...]
        sig_p = 1.0 / (1.0 + jnp.exp(-qv))
        sig_n = 1.0 / (1.0 + jnp.exp(qv))
        grad = (-g1_ref[...] * sig_p + g2_ref[...] * sig_n) * gs[0, 0]
        xin = jnp.concatenate(
            [l_ref[...], c1_ref[...] * cs[0, 0], c2_ref[...] * cs[0, 0],
             grad], axis=1)
        h = _dot(xin, w1[...]) + bb1[...]
        h = _rowln(_relu6(h))
        h = _rowln(_relu6(_dot(h, w2[...]) + bb2[...]))
        y = _dot(h, w3[...]) + bb3[...]
        y_ref[...] = y
        ps = jnp.sum(y, axis=0, keepdims=True)
        pq = jnp.sum(y * y, axis=0, keepdims=True)

        @pl.when(i == 0)
        def _():
            acc[0:1] = ps
            acc[1:2] = pq

        @pl.when(i > 0)
        def _():
            acc[0:1] += ps
            acc[1:2] += pq

        st_ref[...] = acc[...]

    rbfm = pl.BlockSpec((RB, FM), lambda b: (b, 0))
    return pl.pallas_call(
        body,
        grid=(nb,),
        in_specs=[rbfm, rbfm, rbfm, rbfm, rbfm, rbfm,
                  _full((1, 1)), _full((1, 1)),
                  _full((512, 384)), _full((1, 384)),
                  _full((384, 384)), _full((1, 384)),
                  _full((384, FM)), _full((1, FM))],
        out_specs=[rbfm, _full((2, FM))],
        out_shape=[jax.ShapeDtypeStruct((N_VARS, FM), jnp.float32),
                   jax.ShapeDtypeStruct((2, FM), jnp.float32)],
        scratch_shapes=[pltpu.VMEM((2, FM), jnp.float32)],
    )(L, c1, c2, g1, g2, q, cl_scale, g_scale,
      W1, b1, W2, b2, W3, b3)


def _vs(Ln, W1, b1, W2, b2, W3, b3):
    RB = 2000
    nb = N_VARS // RB

    def body(x_ref, w1, bb1, w2, bb2, w3, bb3, y_ref, spp_ref, spn_ref):
        h = _rowln(_relu6(_dot(x_ref[...], w1[...]) + bb1[...]))
        h = _rowln(_relu6(_dot(h, w2[...]) + bb2[...]))
        y = _dot(h, w3[...]) + bb3[...]
        y_ref[...] = y
        spp_ref[...] = jnp.broadcast_to(_softplus(y), (RB, 16))
        spn_ref[...] = jnp.broadcast_to(_softplus(-y), (RB, 16))

    return pl.pallas_call(
        body,
        grid=(nb,),
        in_specs=[pl.BlockSpec((RB, FM), lambda b: (b, 0)),
                  _full((FM, 256)), _full((1, 256)),
                  _full((256, 256)), _full((1, 256)),
                  _full((256, 1)), _full((1, 1))],
        out_specs=[pl.BlockSpec((RB, 1), lambda b: (b, 0)),
                   pl.BlockSpec((RB, 16), lambda b: (b, 0)),
                   pl.BlockSpec((RB, 16), lambda b: (b, 0))],
        out_shape=[jax.ShapeDtypeStruct((N_VARS, 1), jnp.float32),
                   jax.ShapeDtypeStruct((N_VARS, 16), jnp.float32),
                   jax.ShapeDtypeStruct((N_VARS, 16), jnp.float32)],
    )(Ln, W1, b1, W2, b2, W3, b3)


def _pcl(cv16):
    RB = 2000
    nb = N_CLAUSES // RB

    def body(x_ref, o_ref):
        y = jnp.exp(-x_ref[...])
        o_ref[...] = y * (-jnp.log(1.0 - y + EPS))

    return pl.pallas_call(
        body,
        grid=(nb,),
        in_specs=[pl.BlockSpec((RB, 16), lambda b: (b, 0))],
        out_specs=pl.BlockSpec((RB, 16), lambda b: (b, 0)),
        out_shape=jax.ShapeDtypeStruct((N_CLAUSES, 16), jnp.float32),
    )(cv16)


def _loss_final(pg0, pg1):
    def body(a_ref, b_ref, o_ref):
        a = jnp.sum(jnp.sqrt(a_ref[...][:, 0:1] + 1e-6), axis=0,
                    keepdims=True)
        b = jnp.sum(jnp.sqrt(b_ref[...][:, 0:1] + 1e-6), axis=0,
                    keepdims=True)
        o_ref[...] = (a + b) / float(TEST_ROUNDS)

    return pl.pallas_call(
        body,
        out_shape=jax.ShapeDtypeStruct((1, 1), jnp.float32),
    )(pg0, pg1)


def kernel(lit_idx, clause_idx, clause_graph_ids, var_graph_ids, params):
    p = params
    K = 60
    i32 = jnp.int32

    def _pidx(a, fill, k):
        padn = NS * k * EB - a.shape[0]
        return jnp.concatenate(
            [a, jnp.full((padn,), fill, i32)]).reshape(NS, k, EB)

    g_lit = _pidx(lit_idx, 0, K)
    s_cl = _pidx(clause_idx, N_CLAUSES, K)
    g_cl = _pidx(clause_idx, 0, K)
    s_lit = _pidx(lit_idx, N_LITS, K)
    KQ = 20
    g_id = _pidx(jnp.arange(N_CLAUSES, dtype=i32), 0, KQ)
    s_gid = _pidx(clause_graph_ids, N_GRAPHS, KQ)

    def _r(b):
        return jnp.reshape(b, (1, -1))

    vq = p['Vq']
    cu = p['Cu']
    lu = p['Lu']
    vs = p['Vs']
    W1c = cu['W'][0]
    W1l = lu['W'][0]
    lcs = jnp.reshape(p['LC_scale'], (1, 1))
    cls = jnp.reshape(p['CL_scale'], (1, 1))
    gsc = jnp.reshape(p['G_scale'], (1, 1))

    L = jnp.ones((N_VARS, FM), jnp.float32) * p['L_init_scale']
    C = jnp.ones((N_CLAUSES, FM), jnp.float32) * p['C_init_scale']
    pgs = []
    logits = None

    for step in range(TEST_ROUNDS):
        lt = jnp.concatenate([L[:, :64], L[:, 64:]], axis=0)
        lit_tab = jnp.transpose(lt.reshape(N_LITS, 4, 16), (1, 0, 2))
        LCr = _sc_segsum(lit_tab, g_lit, s_cl, N_CLAUSES, "lc_seg")
        lc64 = jnp.transpose(LCr, (1, 0, 2)).reshape(N_CLAUSES, 64)

        q, spp8, spn8 = _vq(L, vq['W'][0], _r(vq['b'][0]), vq['W'][1],
                            _r(vq['b'][1]), vq['W'][2], _r(vq['b'][2]))
        sp_tab = jnp.concatenate([spp8, spn8], axis=1)
        cv8 = _sc_segsum(sp_tab, g_lit, s_cl, N_CLAUSES, "cv_seg")
        closs, closs_sc = _closs(cv8)
        Gt = _sc_segsum(closs_sc, g_cl, s_lit, N_LITS, "gadj_seg")

        C_raw, C_st = _cu(C, closs, lc64, lcs,
                          W1c, _r(cu['b'][0]), cu['W'][1], _r(cu['b'][1]),
                          cu['W'][2], _r(cu['b'][2]))
        Cn, C_sc = _apply_norm(C_raw, C_st, True, "c_norm")
        A = _sc_segsum(C_sc, g_cl, s_lit, N_LITS, "cladj_seg")

        c1 = jnp.transpose(A[:, :N_VARS], (1, 0, 2)).reshape(N_VARS, FM)
        c2 = jnp.transpose(A[:, N_VARS:], (1, 0, 2)).reshape(N_VARS, FM)
        g1 = jnp.transpose(Gt[:, :N_VARS], (1, 0, 2)).reshape(N_VARS, FM)
        g2 = jnp.transpose(Gt[:, N_VARS:], (1, 0, 2)).reshape(N_VARS, FM)
        L_raw, L_st = _lu(
            L, c1, c2, g1, g2, q, cls, gsc,
            W1l, _r(lu['b'][0]), lu['W'][1], _r(lu['b'][1]), lu['W'][2],
            _r(lu['b'][2]))
        (Ln,) = _apply_norm(L_raw, L_st, False, "l_norm")

        logits, spp16, spn16 = _vs(Ln, vs['W'][0], _r(vs['b'][0]),
                                   vs['W'][1], _r(vs['b'][1]), vs['W'][2],
                                   _r(vs['b'][2]))
        sp16 = jnp.concatenate([spp16, spn16], axis=0)[None]
        cv16 = _sc_segsum(sp16, g_lit, s_cl, N_CLAUSES, "cv16_seg")
        pcl16 = _pcl(cv16[0])
        pg = _sc_segsum(pcl16[None], g_id, s_gid, N_GRAPHS, "pg_seg")
        pgs.append(pg[0])
        L, C = Ln, Cn

    loss = _loss_final(pgs[0], pgs[1])
    return logits, jnp.reshape(loss, ()), jnp.array(TEST_ROUNDS - 1, i32)


# speedup vs baseline: 2.3693x; 1.2535x over previous
import functools

import jax
import jax.numpy as jnp
from jax import lax
from jax.experimental import pallas as pl
from jax.experimental.pallas import tpu as pltpu
from jax.experimental.pallas import tpu_sc as plsc

FM = 128
TEST_ROUNDS = 2
N_VARS = 10000
N_LITS = 2 * N_VARS
N_CLAUSES = 40000
N_EDGES = 120000
N_GRAPHS = 64
NORM_EPS = 1e-6
EPS = 1e-8

NC, NS = 2, 16
NW = NC * NS
EB = 128


def _sc_segsum(src, gidx, sidx, n_dst, name):
    G, n_src, CW = src.shape
    K = gidx.shape[1]
    n_acc = ((n_dst + 8) + NS * 64 - 1) // (NS * 64) * (NS * 64)
    zrows = n_acc // NS
    zc = zrows // 8
    frows = (n_dst // (NS * 8)) * 8
    frem = n_dst - frows * NS
    gpc = (G + NC - 1) // NC
    cand = (10, 6, 5, 3, 2, 1) if CW <= 16 else (5, 3, 2, 1)
    F = next(f for f in cand if K % (2 * f) == 0)
    assert n_src >= F * EB and zc % 8 == 0
    NT = K // (2 * F)
    zeros = jnp.zeros((zc, CW), jnp.float32)
    mesh = plsc.VectorSubcoreMesh(core_axis_name="c", subcore_axis_name="s",
                                  num_cores=NC, num_subcores=NS)

    def body(src_h, gidx_h, sidx_h, zero_h, out_h,
             gidx_v, sidx_v, buf0, buf1, zeros_v, acc_sh, semA, semB):
        c = lax.axis_index("c")
        s = lax.axis_index("s")
        pltpu.sync_copy(gidx_h.at[s], gidx_v)
        pltpu.sync_copy(sidx_h.at[s], sidx_v)
        pltpu.sync_copy(zero_h, zeros_v)

        for gi in range(gpc):
            g = gi * NC + c

            @pl.when(g < G)
            def _():
                for z in range(8):
                    pltpu.sync_copy(
                        zeros_v, acc_sh.at[pl.ds(s * zrows + z * zc, zc)])
                plsc.subcore_barrier()

                def fire(tb, buf, sem):
                    for j in range(F):
                        pltpu.async_copy(
                            src_h.at[g].at[gidx_v.at[tb * F + j]],
                            buf.at[pl.ds(j * EB, EB)], sem)

                def drain(buf, sem):
                    pltpu.make_async_copy(
                        src_h.at[g].at[pl.ds(0, F * EB)], buf, sem).wait()

                def scatter(tb, buf):
                    for j in range(F):
                        pltpu.sync_copy(buf.at[pl.ds(j * EB, EB)],
                                        acc_sh.at[sidx_v.at[tb * F + j]],
                                        add=True)

                fire(0, buf0, semA)

                def pair(t, carry):
                    fire(2 * t + 1, buf1, semB)
                    drain(buf0, semA)
                    scatter(2 * t, buf0)

                    @pl.when(t + 1 < NT)
                    def _():
                        fire(2 * t + 2, buf0, semA)

                    drain(buf1, semB)
                    scatter(2 * t + 1, buf1)
                    return carry

                lax.fori_loop(0, NT, pair, 0)
                plsc.subcore_barrier()
                if frows > 0:
                    pltpu.sync_copy(acc_sh.at[pl.ds(s * frows, frows)],
                                    out_h.at[g].at[pl.ds(s * frows, frows)])
                if frem > 0:
                    @pl.when(s == 0)
                    def _():
                        pltpu.sync_copy(
                            acc_sh.at[pl.ds(NS * frows, frem)],
                            out_h.at[g].at[pl.ds(NS * frows, frem)])
                plsc.subcore_barrier()

    return pl.kernel(
        body,
        out_type=jax.ShapeDtypeStruct((G, n_dst, CW), jnp.float32),
        mesh=mesh,
        scratch_types=[
            pltpu.VMEM((K, EB), jnp.int32),
            pltpu.VMEM((K, EB), jnp.int32),
            pltpu.VMEM((F * EB, CW), jnp.float32),
            pltpu.VMEM((F * EB, CW), jnp.float32),
            pltpu.VMEM((zc, CW), jnp.float32),
            pltpu.VMEM_SHARED((n_acc, CW), jnp.float32),
            pltpu.SemaphoreType.DMA,
            pltpu.SemaphoreType.DMA,
        ],
        name=name,
        compiler_params=pltpu.CompilerParams(use_tc_tiling_on_sc=False),
    )(src, gidx, sidx, zeros)


def _relu6(x):
    return jnp.minimum(jnp.maximum(x, 0.0), 6.0)


def _rowln(x):
    m = jnp.mean(x, axis=-1, keepdims=True)
    xm = x - m
    v = jnp.mean(xm * xm, axis=-1, keepdims=True)
    return xm * lax.rsqrt(v + 1e-6)


def _softplus(x):
    return jnp.maximum(x, 0.0) + jnp.log1p(jnp.exp(-jnp.abs(x)))


def _dot(a, b):
    return jnp.dot(a, b, preferred_element_type=jnp.float32)


def _full(shape):
    return pl.BlockSpec(shape, lambda b: tuple(0 for _ in shape))


def _vq(L, W1, b1, W2, b2, W3, b3):
    RB = 2000
    nb = N_VARS // RB

    def body(x_ref, w1, bb1, w2, bb2, w3, bb3, q_ref, spp_ref, spn_ref):
        h = _relu6(_dot(x_ref[...], w1[...]) + bb1[...])
        h = _relu6(_dot(h, w2[...]) + bb2[...])
        q = _dot(h, w3[...]) + bb3[...]
        q_ref[...] = q
        spp = _softplus(q)
        spn = _softplus(-q)
        for g in range(8):
            spp_ref[g] = spp[:, g * 16:(g + 1) * 16]
            spn_ref[g] = spn[:, g * 16:(g + 1) * 16]

    return pl.pallas_call(
        body,
        grid=(nb,),
        in_specs=[pl.BlockSpec((RB, FM), lambda b: (b, 0)),
                  _full((FM, FM)), _full((1, FM)),
                  _full((FM, FM)), _full((1, FM)),
                  _full((FM, FM)), _full((1, FM))],
        out_specs=[pl.BlockSpec((RB, FM), lambda b: (b, 0)),
                   pl.BlockSpec((8, RB, 16), lambda b: (0, b, 0)),
                   pl.BlockSpec((8, RB, 16), lambda b: (0, b, 0))],
        out_shape=[jax.ShapeDtypeStruct((N_VARS, FM), jnp.float32),
                   jax.ShapeDtypeStruct((8, N_VARS, 16), jnp.float32),
                   jax.ShapeDtypeStruct((8, N_VARS, 16), jnp.float32)],
    )(L, W1, b1, W2, b2, W3, b3)


def _closs(cv8):
    RB = 2000
    nb = N_CLAUSES // RB

    def body(cv_ref, full_ref, sc_ref):
        y = jnp.exp(-jnp.concatenate([cv_ref[g] for g in range(8)], axis=1))
        full_ref[...] = y
        for k in range(4):
            sc_ref[k] = y[:, k * 32:(k + 1) * 32]

    return pl.pallas_call(
        body,
        grid=(nb,),
        in_specs=[pl.BlockSpec((8, RB, 16), lambda b: (0, b, 0))],
        out_specs=[pl.BlockSpec((RB, FM), lambda b: (b, 0)),
                   pl.BlockSpec((4, RB, 32), lambda b: (0, b, 0))],
        out_shape=[jax.ShapeDtypeStruct((N_CLAUSES, FM), jnp.float32),
                   jax.ShapeDtypeStruct((4, N_CLAUSES, 32), jnp.float32)],
    )(cv8)


def _cu(C, closs, lc, lc_scale, W1, b1, W2, b2, W3, b3):
    RB = 2000
    nb = N_CLAUSES // RB

    def body(c_ref, cl_ref, lc_ref, sc, w1, bb1, w2, bb2,
             w3, bb3, y_ref, st_ref, acc):
        i = pl.program_id(0)
        xin = jnp.concatenate(
            [c_ref[...], cl_ref[...], lc_ref[...] * sc[0, 0]], axis=1)
        h = _dot(xin, w1[...]) + bb1[...]
        h = _rowln(_relu6(h))
        h = _rowln(_relu6(_dot(h, w2[...]) + bb2[...]))
        y = _dot(h, w3[...]) + bb3[...]
        y_ref[...] = y
        ps = jnp.sum(y, axis=0, keepdims=True)
        pq = jnp.sum(y * y, axis=0, keepdims=True)

        @pl.when(i == 0)
        def _():
            acc[0:1] = ps
            acc[1:2] = pq

        @pl.when(i > 0)
        def _():
            acc[0:1] += ps
            acc[1:2] += pq

        st_ref[...] = acc[...]

    return pl.pallas_call(
        body,
        grid=(nb,),
        in_specs=[pl.BlockSpec((RB, FM), lambda b: (b, 0)),
                  pl.BlockSpec((RB, FM), lambda b: (b, 0)),
                  pl.BlockSpec((RB, 64), lambda b: (b, 0)),
                  _full((1, 1)),
                  _full((320, 256)), _full((1, 256)),
                  _full((256, 256)), _full((1, 256)),
                  _full((256, FM)), _full((1, FM))],
        out_specs=[pl.BlockSpec((RB, FM), lambda b: (b, 0)),
                   _full((2, FM))],
        out_shape=[jax.ShapeDtypeStruct((N_CLAUSES, FM), jnp.float32),
                   jax.ShapeDtypeStruct((2, FM), jnp.float32)],
        scratch_shapes=[pltpu.VMEM((2, FM), jnp.float32)],
    )(C, closs, lc, lc_scale, W1, b1, W2, b2, W3, b3)


def _apply_norm(x, stats, emit_sc, name):
    n_rows = x.shape[0]
    RB = 2000
    nb = n_rows // RB
    fn = float(n_rows)

    def body(x_ref, st, y_ref, *maybe_sc):
        m = st[0:1] / fn
        ex2 = st[1:2] / fn
        v = ex2 - m * m
        y = (x_ref[...] - m) * lax.rsqrt(v + NORM_EPS)
        y_ref[...] = y
        if emit_sc:
            for k in range(4):
                maybe_sc[0][k] = y[:, k * 32:(k + 1) * 32]

    out_specs = [pl.BlockSpec((RB, FM), lambda b: (b, 0))]
    out_shape = [jax.ShapeDtypeStruct((n_rows, FM), jnp.float32)]
    if emit_sc:
        out_specs.append(pl.BlockSpec((4, RB, 32), lambda b: (0, b, 0)))
        out_shape.append(jax.ShapeDtypeStruct((4, n_rows, 32), jnp.float32))
    return pl.pallas_call(
        body,
        grid=(nb,),
        in_specs=[pl.BlockSpec((RB, FM), lambda b: (b, 0)), _full((2, FM))],
        out_specs=out_specs,
        out_shape=out_shape,
        name=name,
    )(x, stats)


def _lu(L, c1, c2, g1, g2, q, cl_scale, g_scale,
        W1, b1, W2, b2, W3, b3):
    RB = 2000
    nb = N_VARS // RB

    def body(l_ref, c1_ref, c2_ref, g1_ref, g2_ref, q_ref, cs, gs,
             w1, bb1, w2, bb2, w3, bb3,
             y_ref, st_ref, acc):
        i = pl.program_id(0)
        qv = q_ref[...]
        sig_p = 1.0 / (1.0 + jnp.exp(-qv))
        sig_n = 1.0 / (1.0 + jnp.exp(qv))
        grad = (-g1_ref[...] * sig_p + g2_ref[...] * sig_n) * gs[0, 0]
        xin = jnp.concatenate(
            [l_ref[...], c1_ref[...] * cs[0, 0], c2_ref[...] * cs[0, 0],
             grad], axis=1)
        h = _dot(xin, w1[...]) + bb1[...]
        h = _rowln(_relu6(h))
        h = _rowln(_relu6(_dot(h, w2[...]) + bb2[...]))
        y = _dot(h, w3[...]) + bb3[...]
        y_ref[...] = y
        ps = jnp.sum(y, axis=0, keepdims=True)
        pq = jnp.sum(y * y, axis=0, keepdims=True)

        @pl.when(i == 0)
        def _():
            acc[0:1] = ps
            acc[1:2] = pq

        @pl.when(i > 0)
        def _():
            acc[0:1] += ps
            acc[1:2] += pq

        st_ref[...] = acc[...]

    rbfm = pl.BlockSpec((RB, FM), lambda b: (b, 0))
    return pl.pallas_call(
        body,
        grid=(nb,),
        in_specs=[rbfm, rbfm, rbfm, rbfm, rbfm, rbfm,
                  _full((1, 1)), _full((1, 1)),
                  _full((512, 384)), _full((1, 384)),
                  _full((384, 384)), _full((1, 384)),
                  _full((384, FM)), _full((1, FM))],
        out_specs=[rbfm, _full((2, FM))],
        out_shape=[jax.ShapeDtypeStruct((N_VARS, FM), jnp.float32),
                   jax.ShapeDtypeStruct((2, FM), jnp.float32)],
        scratch_shapes=[pltpu.VMEM((2, FM), jnp.float32)],
    )(L, c1, c2, g1, g2, q, cl_scale, g_scale,
      W1, b1, W2, b2, W3, b3)


def _vs(Ln, W1, b1, W2, b2, W3, b3):
    RB = 2000
    nb = N_VARS // RB

    def body(x_ref, w1, bb1, w2, bb2, w3, bb3, y_ref, spp_ref, spn_ref):
        h = _rowln(_relu6(_dot(x_ref[...], w1[...]) + bb1[...]))
        h = _rowln(_relu6(_dot(h, w2[...]) + bb2[...]))
        y = _dot(h, w3[...]) + bb3[...]
        y_ref[...] = y
        spp_ref[...] = jnp.broadcast_to(_softplus(y), (RB, 16))
        spn_ref[...] = jnp.broadcast_to(_softplus(-y), (RB, 16))

    return pl.pallas_call(
        body,
        grid=(nb,),
        in_specs=[pl.BlockSpec((RB, FM), lambda b: (b, 0)),
                  _full((FM, 256)), _full((1, 256)),
                  _full((256, 256)), _full((1, 256)),
                  _full((256, 1)), _full((1, 1))],
        out_specs=[pl.BlockSpec((RB, 1), lambda b: (b, 0)),
                   pl.BlockSpec((RB, 16), lambda b: (b, 0)),
                   pl.BlockSpec((RB, 16), lambda b: (b, 0))],
        out_shape=[jax.ShapeDtypeStruct((N_VARS, 1), jnp.float32),
                   jax.ShapeDtypeStruct((N_VARS, 16), jnp.float32),
                   jax.ShapeDtypeStruct((N_VARS, 16), jnp.float32)],
    )(Ln, W1, b1, W2, b2, W3, b3)


def _pcl(cv16):
    RB = 2000
    nb = N_CLAUSES // RB

    def body(x_ref, o_ref):
        y = jnp.exp(-x_ref[...])
        o_ref[...] = y * (-jnp.log(1.0 - y + EPS))

    return pl.pallas_call(
        body,
        grid=(nb,),
        in_specs=[pl.BlockSpec((RB, 16), lambda b: (b, 0))],
        out_specs=pl.BlockSpec((RB, 16), lambda b: (b, 0)),
        out_shape=jax.ShapeDtypeStruct((N_CLAUSES, 16), jnp.float32),
    )(cv16)


def _loss_final(pg0, pg1):
    def body(a_ref, b_ref, o_ref):
        a = jnp.sum(jnp.sqrt(a_ref[...][:, 0:1] + 1e-6), axis=0,
                    keepdims=True)
        b = jnp.sum(jnp.sqrt(b_ref[...][:, 0:1] + 1e-6), axis=0,
                    keepdims=True)
        o_ref[...] = (a + b) / float(TEST_ROUNDS)

    return pl.pallas_call(
        body,
        out_shape=jax.ShapeDtypeStruct((1, 1), jnp.float32),
    )(pg0, pg1)


def kernel(lit_idx, clause_idx, clause_graph_ids, var_graph_ids, params):
    p = params
    K = 60
    i32 = jnp.int32

    def _pidx(a, fill, k):
        padn = NS * k * EB - a.shape[0]
        return jnp.concatenate(
            [a, jnp.full((padn,), fill, i32)]).reshape(NS, k, EB)

    g_lit = _pidx(lit_idx, 0, K)
    s_cl = _pidx(clause_idx, N_CLAUSES, K)
    g_cl = _pidx(clause_idx, 0, K)
    s_lit = _pidx(lit_idx, N_LITS, K)
    KQ = 24
    g_id = _pidx(jnp.arange(N_CLAUSES, dtype=i32), 0, KQ)
    s_gid = _pidx(clause_graph_ids, N_GRAPHS, KQ)

    def _r(b):
        return jnp.reshape(b, (1, -1))

    vq = p['Vq']
    cu = p['Cu']
    lu = p['Lu']
    vs = p['Vs']
    W1c = cu['W'][0]
    W1l = lu['W'][0]
    lcs = jnp.reshape(p['LC_scale'], (1, 1))
    cls = jnp.reshape(p['CL_scale'], (1, 1))
    gsc = jnp.reshape(p['G_scale'], (1, 1))

    L = jnp.ones((N_VARS, FM), jnp.float32) * p['L_init_scale']
    C = jnp.ones((N_CLAUSES, FM), jnp.float32) * p['C_init_scale']
    pgs = []
    logits = None

    for step in range(TEST_ROUNDS):
        lt = jnp.concatenate([L[:, :64], L[:, 64:]], axis=0)
        lit_tab = jnp.transpose(lt.reshape(N_LITS, 4, 16), (1, 0, 2))
        LCr = _sc_segsum(lit_tab, g_lit, s_cl, N_CLAUSES, "lc_seg")
        lc64 = jnp.transpose(LCr, (1, 0, 2)).reshape(N_CLAUSES, 64)

        q, spp8, spn8 = _vq(L, vq['W'][0], _r(vq['b'][0]), vq['W'][1],
                            _r(vq['b'][1]), vq['W'][2], _r(vq['b'][2]))
        sp_tab = jnp.concatenate([spp8, spn8], axis=1)
        cv8 = _sc_segsum(sp_tab, g_lit, s_cl, N_CLAUSES, "cv_seg")
        closs, closs_sc = _closs(cv8)
        Gt = _sc_segsum(closs_sc, g_cl, s_lit, N_LITS, "gadj_seg")

        C_raw, C_st = _cu(C, closs, lc64, lcs,
                          W1c, _r(cu['b'][0]), cu['W'][1], _r(cu['b'][1]),
                          cu['W'][2], _r(cu['b'][2]))
        Cn, C_sc = _apply_norm(C_raw, C_st, True, "c_norm")
        A = _sc_segsum(C_sc, g_cl, s_lit, N_LITS, "cladj_seg")

        c1 = jnp.transpose(A[:, :N_VARS], (1, 0, 2)).reshape(N_VARS, FM)
        c2 = jnp.transpose(A[:, N_VARS:], (1, 0, 2)).reshape(N_VARS, FM)
        g1 = jnp.transpose(Gt[:, :N_VARS], (1, 0, 2)).reshape(N_VARS, FM)
        g2 = jnp.transpose(Gt[:, N_VARS:], (1, 0, 2)).reshape(N_VARS, FM)
        L_raw, L_st = _lu(
            L, c1, c2, g1, g2, q, cls, gsc,
            W1l, _r(lu['b'][0]), lu['W'][1], _r(lu['b'][1]), lu['W'][2],
            _r(lu['b'][2]))
        (Ln,) = _apply_norm(L_raw, L_st, False, "l_norm")

        logits, spp16, spn16 = _vs(Ln, vs['W'][0], _r(vs['b'][0]),
                                   vs['W'][1], _r(vs['b'][1]), vs['W'][2],
                                   _r(vs['b'][2]))
        sp16 = jnp.concatenate([spp16, spn16], axis=0)[None]
        cv16 = _sc_segsum(sp16, g_lit, s_cl, N_CLAUSES, "cv16_seg")
        pcl16 = _pcl(cv16[0])
        pg = _sc_segsum(pcl16[None], g_id, s_gid, N_GRAPHS, "pg_seg")
        pgs.append(pg[0])
        L, C = Ln, Cn

    loss = _loss_final(pgs[0], pgs[1])
    return logits, jnp.reshape(loss, ()), jnp.array(TEST_ROUNDS - 1, i32)
